# baseline (speedup 1.0000x reference)
import functools

import jax
import jax.numpy as jnp
from jax import lax
from jax.experimental import pallas as pl
from jax.experimental.pallas import tpu as pltpu
from jax.experimental.pallas import tpu_sc as plsc

NC = 2
NS = 16
L = 16
NW = NC * NS

N = 10000
K = 256
C = 40
RPT = 768
NP = NS * RPT
D = 128
DH = D // 2
NGRP = DH // L
XRT = 632


def _make_sc_kernel():
    mesh = plsc.VectorSubcoreMesh(core_axis_name="c", subcore_axis_name="s")

    @functools.partial(
        pl.kernel,
        mesh=mesh,
        compiler_params=pltpu.CompilerParams(use_tc_tiling_on_sc=False),
        out_type=jax.ShapeDtypeStruct((NC, 2, NP, DH), jnp.float32),
        scratch_types=[
            pltpu.VMEM((2, K), jnp.int32),
            pltpu.VMEM((2, K), jnp.int32),
            pltpu.VMEM((2, K), jnp.int32),
            pltpu.VMEM((2, K), jnp.int32),
            pltpu.VMEM((K,), jnp.float32),
            pltpu.VMEM((K,), jnp.float32),
            pltpu.VMEM((K,), jnp.float32),
            pltpu.VMEM((K,), jnp.float32),
            pltpu.VMEM((K, DH), jnp.float32),
            pltpu.VMEM((K, DH), jnp.float32),
            pltpu.VMEM_SHARED((N, DH), jnp.float32),
            pltpu.VMEM_SHARED((NP, DH), jnp.float32),
            pltpu.SemaphoreType.DMA,
            pltpu.SemaphoreType.DMA,
            pltpu.SemaphoreType.DMA,
            pltpu.SemaphoreType.DMA,
            pltpu.SemaphoreType.DMA,
            pltpu.SemaphoreType.DMA,
            pltpu.SemaphoreType.DMA,
        ],
    )
    def sc_kernel(x0_hbm, x1_hbm, pk_hbm, wk_hbm, out_hbm,
                  e30, e31, e32, e33, wr0, wr1, wr2, wr3,
                  rows0_v, rows1_v, xs, acc,
                  si0, si1, si2, si3, sg0, sg1, sstage):
        cid = lax.axis_index("c")
        sid = lax.axis_index("s")
        wid = cid * NS + sid
        e3 = (e30, e31, e32, e33)
        wr = (wr0, wr1, wr2, wr3)
        sem_i = (si0, si1, si2, si3)
        bufs = (rows0_v, rows1_v)
        sem_g = (sg0, sg1)
        base = sid * RPT
        xr0 = sid * XRT
        lsz = N - (NS - 1) * XRT
        sp = 128
        stage_szs = (sp, sp, sp, sp, XRT - 4 * sp)
        stage_szs_last = (sp, sp, sp, sp, lsz - 4 * sp)

        for p, x_hbm in enumerate((x0_hbm, x1_hbm)):
            @pl.when(sid < NS - 1)
            def _():
                xoff = 0
                for sz in stage_szs:
                    pltpu.async_copy(x_hbm.at[pl.ds(xr0 + xoff, sz)],
                                     xs.at[pl.ds(xr0 + xoff, sz)], sstage)
                    xoff += sz

            @pl.when(sid == NS - 1)
            def _():
                xoff = 0
                for sz in stage_szs_last:
                    pltpu.async_copy(x_hbm.at[pl.ds(xr0 + xoff, sz)],
                                     xs.at[pl.ds(xr0 + xoff, sz)], sstage)
                    xoff += sz

            def _zero_row(r, _):
                for g in range(NGRP):
                    rows0_v[r, pl.ds(g * L, L)] = jnp.zeros((L,), jnp.float32)
                return 0

            lax.fori_loop(0, K, _zero_row, 0)
            for t in range(RPT // K):
                pltpu.sync_copy(rows0_v, acc.at[pl.ds(base + t * K, K)])

            @pl.when(sid < NS - 1)
            def _():
                xoff = 0
                for sz in stage_szs:
                    pltpu.make_async_copy(x_hbm.at[pl.ds(xr0 + xoff, sz)],
                                          xs.at[pl.ds(xr0 + xoff, sz)],
                                          sstage).wait()
                    xoff += sz

            @pl.when(sid == NS - 1)
            def _():
                xoff = 0
                for sz in stage_szs_last:
                    pltpu.make_async_copy(x_hbm.at[pl.ds(xr0 + xoff, sz)],
                                          xs.at[pl.ds(xr0 + xoff, sz)],
                                          sstage).wait()
                    xoff += sz

            plsc.subcore_barrier()

            for d in range(4):
                pltpu.async_copy(pk_hbm.at[wid, d], e3[d], sem_i[d])
                pltpu.async_copy(wk_hbm.at[wid, d], wr[d], sem_i[d])
            for b in range(2):
                pltpu.make_async_copy(pk_hbm.at[wid, b], e3[b],
                                      sem_i[b]).wait()
                pltpu.make_async_copy(wk_hbm.at[wid, b], wr[b],
                                      sem_i[b]).wait()
                pltpu.async_copy(xs.at[e3[b].at[0]], bufs[b], sem_g[b])

            def _quad(it, _):
                for b in range(4):
                    c = it * 4 + b
                    rows_v = bufs[b % 2]
                    e3b = e3[b]
                    wrb = wr[b]
                    pltpu.make_async_copy(xs.at[e3b.at[0]], rows_v,
                                          sem_g[b % 2]).wait()

                    def _scale_block(bi, _):
                        wvec = wrb[pl.ds(bi * L, L)]
                        for j in range(L):
                            wv = jnp.full((L,), wvec[j], jnp.float32)
                            e = bi * L + j
                            for g in range(NGRP):
                                sl = pl.ds(g * L, L)
                                rows_v[e, sl] = rows_v[e, sl] * wv
                        return 0

                    lax.fori_loop(0, K // L, _scale_block, 0)
                    pltpu.sync_copy(rows_v, acc.at[e3b.at[1]], add=True)

                    nc4 = c + 4

                    @pl.when(nc4 < C)
                    def _():
                        pltpu.async_copy(pk_hbm.at[wid, nc4], e3b, sem_i[b])
                        pltpu.async_copy(wk_hbm.at[wid, nc4], wrb, sem_i[b])

                    nc2 = c + 2

                    @pl.when(nc2 < C)
                    def _():
                        b2 = (b + 2) % 4
                        pltpu.make_async_copy(pk_hbm.at[wid, nc2], e3[b2],
                                              sem_i[b2]).wait()
                        pltpu.make_async_copy(wk_hbm.at[wid, nc2], wr[b2],
                                              sem_i[b2]).wait()
                        pltpu.async_copy(xs.at[e3[b2].at[0]], rows_v,
                                         sem_g[b % 2])
                return 0

            lax.fori_loop(0, C // 4, _quad, 0)
            plsc.subcore_barrier()

            nt = RPT // K
            for t in range(nt):
                bb = bufs[t % 2]
                pltpu.sync_copy(acc.at[pl.ds(base + t * K, K)], bb)
                pltpu.async_copy(bb, out_hbm.at[cid, p, pl.ds(base + t * K, K)],
                                 sem_g[t % 2])
                if t >= 1:
                    pb = bufs[(t - 1) % 2]
                    pltpu.make_async_copy(
                        pb, out_hbm.at[cid, p, pl.ds(base + (t - 1) * K, K)],
                        sem_g[(t - 1) % 2]).wait()
            pltpu.make_async_copy(
                bufs[(nt - 1) % 2],
                out_hbm.at[cid, p, pl.ds(base + (nt - 1) * K, K)],
                sem_g[(nt - 1) % 2]).wait()
            if p == 0:
                plsc.subcore_barrier()

    return sc_kernel


def _tc_body(p_ref, w_ref, b_ref, o_ref):
    a = jnp.concatenate([p_ref[0, 0] + p_ref[1, 0],
                         p_ref[0, 1] + p_ref[1, 1]], axis=-1)
    y = lax.dot_general(a, w_ref[...], (((1,), (1,)), ((), ())),
                        preferred_element_type=jnp.float32,
                        precision=lax.Precision.HIGHEST)
    o_ref[...] = y + b_ref[...]


def kernel(x, edge_index, edge_weight, W, b):
    n = x.shape[0]
    e = edge_weight.shape[0]
    ep = NW * C * K
    src = jnp.pad(edge_index[1].astype(jnp.int32), (0, ep - e)).reshape(NW, C, K)
    dst = jnp.pad(edge_index[0].astype(jnp.int32), (0, ep - e)).reshape(NW, C, K)
    w = jnp.pad(edge_weight, (0, ep - e)).reshape(NW, C, K)
    pk = jnp.stack([src, dst], axis=2)
    x0 = x[:, :DH]
    x1 = x[:, DH:]

    partials = _make_sc_kernel()(x0, x1, pk, w)

    rblk = 400
    out = pl.pallas_call(
        _tc_body,
        grid=(n // rblk,),
        in_specs=[
            pl.BlockSpec((NC, 2, rblk, DH), lambda i: (0, 0, i, 0)),
            pl.BlockSpec((D, D), lambda i: (0, 0)),
            pl.BlockSpec((1, D), lambda i: (0, 0)),
        ],
        out_specs=pl.BlockSpec((rblk, D), lambda i: (i, 0)),
        out_shape=jax.ShapeDtypeStruct((n, D), jnp.float32),
    )(partials, W, b.reshape(1, D))
    return out


# speedup vs baseline: 1.4805x; 1.4805x over previous
import functools

import jax
import jax.numpy as jnp
from jax import lax
from jax.experimental import pallas as pl
from jax.experimental.pallas import tpu as pltpu
from jax.experimental.pallas import tpu_sc as plsc

NC = 2
NS = 16
L = 16
NW = NC * NS

N = 10000
K = 128
C = 80
RPT = 640
NP = NS * RPT
D = 128
DH = D // 2
NGRP = DH // L
XRT = 632


def _make_sc_kernel():
    mesh = plsc.VectorSubcoreMesh(core_axis_name="c", subcore_axis_name="s")

    @functools.partial(
        pl.kernel,
        mesh=mesh,
        compiler_params=pltpu.CompilerParams(use_tc_tiling_on_sc=False),
        out_type=jax.ShapeDtypeStruct((NC, 2, NP, DH), jnp.float32),
        scratch_types=[
            pltpu.VMEM((2, K), jnp.int32),
            pltpu.VMEM((2, K), jnp.int32),
            pltpu.VMEM((2, K), jnp.int32),
            pltpu.VMEM((2, K), jnp.int32),
            pltpu.VMEM((K,), jnp.float32),
            pltpu.VMEM((K,), jnp.float32),
            pltpu.VMEM((K,), jnp.float32),
            pltpu.VMEM((K,), jnp.float32),
            pltpu.VMEM((K, DH), jnp.float32),
            pltpu.VMEM((K, DH), jnp.float32),
            pltpu.VMEM_SHARED((N, DH), jnp.float32),
            pltpu.VMEM_SHARED((NP, DH), jnp.float32),
            pltpu.SemaphoreType.DMA,
            pltpu.SemaphoreType.DMA,
            pltpu.SemaphoreType.DMA,
            pltpu.SemaphoreType.DMA,
            pltpu.SemaphoreType.DMA,
            pltpu.SemaphoreType.DMA,
            pltpu.SemaphoreType.DMA,
        ],
    )
    def sc_kernel(x0_hbm, x1_hbm, pk_hbm, wk_hbm, out_hbm,
                  e30, e31, e32, e33, wr0, wr1, wr2, wr3,
                  rows0_v, rows1_v, xs, acc,
                  si0, si1, si2, si3, sg0, sg1, sstage):
        cid = lax.axis_index("c")
        sid = lax.axis_index("s")
        wid = cid * NS + sid
        e3 = (e30, e31, e32, e33)
        wr = (wr0, wr1, wr2, wr3)
        sem_i = (si0, si1, si2, si3)
        bufs = (rows0_v, rows1_v)
        sem_g = (sg0, sg1)
        base = sid * RPT
        xr0 = sid * XRT
        lsz = N - (NS - 1) * XRT
        sp = 128
        stage_szs = (sp, sp, sp, sp, XRT - 4 * sp)
        stage_szs_last = (sp, sp, sp, sp, lsz - 4 * sp)

        for p, x_hbm in enumerate((x0_hbm, x1_hbm)):
            @pl.when(sid < NS - 1)
            def _():
                xoff = 0
                for sz in stage_szs:
                    pltpu.async_copy(x_hbm.at[pl.ds(xr0 + xoff, sz)],
                                     xs.at[pl.ds(xr0 + xoff, sz)], sstage)
                    xoff += sz

            @pl.when(sid == NS - 1)
            def _():
                xoff = 0
                for sz in stage_szs_last:
                    pltpu.async_copy(x_hbm.at[pl.ds(xr0 + xoff, sz)],
                                     xs.at[pl.ds(xr0 + xoff, sz)], sstage)
                    xoff += sz

            def _zero_row(r, _):
                for g in range(NGRP):
                    rows0_v[r, pl.ds(g * L, L)] = jnp.zeros((L,), jnp.float32)
                return 0

            lax.fori_loop(0, K, _zero_row, 0)
            for t in range(RPT // K):
                pltpu.sync_copy(rows0_v, acc.at[pl.ds(base + t * K, K)])

            @pl.when(sid < NS - 1)
            def _():
                xoff = 0
                for sz in stage_szs:
                    pltpu.make_async_copy(x_hbm.at[pl.ds(xr0 + xoff, sz)],
                                          xs.at[pl.ds(xr0 + xoff, sz)],
                                          sstage).wait()
                    xoff += sz

            @pl.when(sid == NS - 1)
            def _():
                xoff = 0
                for sz in stage_szs_last:
                    pltpu.make_async_copy(x_hbm.at[pl.ds(xr0 + xoff, sz)],
                                          xs.at[pl.ds(xr0 + xoff, sz)],
                                          sstage).wait()
                    xoff += sz

            plsc.subcore_barrier()

            for d in range(4):
                pltpu.async_copy(pk_hbm.at[wid, d], e3[d], sem_i[d])
                pltpu.async_copy(wk_hbm.at[wid, d], wr[d], sem_i[d])
            for b in range(2):
                pltpu.make_async_copy(pk_hbm.at[wid, b], e3[b],
                                      sem_i[b]).wait()
                pltpu.make_async_copy(wk_hbm.at[wid, b], wr[b],
                                      sem_i[b]).wait()
                pltpu.async_copy(xs.at[e3[b].at[0]], bufs[b], sem_g[b])

            def _quad(it, _):
                for b in range(4):
                    c = it * 4 + b
                    rows_v = bufs[b % 2]
                    e3b = e3[b]
                    wrb = wr[b]
                    pltpu.make_async_copy(xs.at[e3b.at[0]], rows_v,
                                          sem_g[b % 2]).wait()

                    for bi in range(K // L):
                        wvec = wrb[pl.ds(bi * L, L)]
                        for j in range(L):
                            wv = jnp.full((L,), wvec[j], jnp.float32)
                            e = bi * L + j
                            for g in range(NGRP):
                                sl = pl.ds(g * L, L)
                                rows_v[e, sl] = rows_v[e, sl] * wv

                    pltpu.sync_copy(rows_v, acc.at[e3b.at[1]], add=True)

                    nc4 = c + 4

                    @pl.when(nc4 < C)
                    def _():
                        pltpu.async_copy(pk_hbm.at[wid, nc4], e3b, sem_i[b])
                        pltpu.async_copy(wk_hbm.at[wid, nc4], wrb, sem_i[b])

                    nc2 = c + 2

                    @pl.when(nc2 < C)
                    def _():
                        b2 = (b + 2) % 4
                        pltpu.make_async_copy(pk_hbm.at[wid, nc2], e3[b2],
                                              sem_i[b2]).wait()
                        pltpu.make_async_copy(wk_hbm.at[wid, nc2], wr[b2],
                                              sem_i[b2]).wait()
                        pltpu.async_copy(xs.at[e3[b2].at[0]], rows_v,
                                         sem_g[b % 2])
                return 0

            lax.fori_loop(0, C // 4, _quad, 0)
            plsc.subcore_barrier()

            nt = RPT // K
            for t in range(nt):
                bb = bufs[t % 2]
                pltpu.sync_copy(acc.at[pl.ds(base + t * K, K)], bb)
                pltpu.async_copy(bb, out_hbm.at[cid, p, pl.ds(base + t * K, K)],
                                 sem_g[t % 2])
                if t >= 1:
                    pb = bufs[(t - 1) % 2]
                    pltpu.make_async_copy(
                        pb, out_hbm.at[cid, p, pl.ds(base + (t - 1) * K, K)],
                        sem_g[(t - 1) % 2]).wait()
            pltpu.make_async_copy(
                bufs[(nt - 1) % 2],
                out_hbm.at[cid, p, pl.ds(base + (nt - 1) * K, K)],
                sem_g[(nt - 1) % 2]).wait()
            if p == 0:
                plsc.subcore_barrier()

    return sc_kernel


def _tc_body(p_ref, w_ref, b_ref, o_ref):
    a = jnp.concatenate([p_ref[0, 0] + p_ref[1, 0],
                         p_ref[0, 1] + p_ref[1, 1]], axis=-1)
    y = lax.dot_general(a, w_ref[...], (((1,), (1,)), ((), ())),
                        preferred_element_type=jnp.float32,
                        precision=lax.Precision.HIGHEST)
    o_ref[...] = y + b_ref[...]


def kernel(x, edge_index, edge_weight, W, b):
    n = x.shape[0]
    e = edge_weight.shape[0]
    ep = NW * C * K
    src = jnp.pad(edge_index[1].astype(jnp.int32), (0, ep - e)).reshape(NW, C, K)
    dst = jnp.pad(edge_index[0].astype(jnp.int32), (0, ep - e)).reshape(NW, C, K)
    w = jnp.pad(edge_weight, (0, ep - e)).reshape(NW, C, K)
    pk = jnp.stack([src, dst], axis=2)
    x0 = x[:, :DH]
    x1 = x[:, DH:]

    partials = _make_sc_kernel()(x0, x1, pk, w)

    rblk = 400
    out = pl.pallas_call(
        _tc_body,
        grid=(n // rblk,),
        in_specs=[
            pl.BlockSpec((NC, 2, rblk, DH), lambda i: (0, 0, i, 0)),
            pl.BlockSpec((D, D), lambda i: (0, 0)),
            pl.BlockSpec((1, D), lambda i: (0, 0)),
        ],
        out_specs=pl.BlockSpec((rblk, D), lambda i: (i, 0)),
        out_shape=jax.ShapeDtypeStruct((n, D), jnp.float32),
    )(partials, W, b.reshape(1, D))
    return out
